# baseline (speedup 1.0000x reference)
import jax
import jax.numpy as jnp
from jax.experimental import pallas as pl

_B = 2
_NC = 90
_A = 5456
_SBLK = 496
_K = 5000


def _gmm_kernel(cls_ref, box_ref, ca_ref, cua_ref, cue_ref, ba_ref, bua_ref, bue_ref):
    base = pl.program_id(1) * _SBLK

    def mix(x, swap_l4=False):
        m0, m1, m2 = x[0], x[1], x[2]
        v0, v1, v2 = x[3], x[4], x[5]
        w0, w1, w2 = x[6], x[7], x[8]
        mx = jnp.maximum(jnp.maximum(w0, w1), w2)
        e0 = jnp.exp(w0 - mx)
        e1 = jnp.exp(w1 - mx)
        e2 = jnp.exp(w2 - mx)
        s = (e0 + e1) + e2
        t0 = e0 / s
        t1 = e1 / s
        t2 = e2 / s
        p0, p1, p2 = t0 * m0, t1 * m1, t2 * m2
        wm = (p0 + p1) + p2
        if swap_l4:
            row = base + jax.lax.broadcasted_iota(jnp.int32, wm.shape, 0)
            wm = jnp.where(row >= _A - 16, (p0 + p2) + p1, wm)
        ua = (t0 * jax.nn.sigmoid(v0) + t1 * jax.nn.sigmoid(v1)) \
            + t2 * jax.nn.sigmoid(v2)
        d0 = m0 - wm
        d1 = m1 - wm
        d2 = m2 - wm
        ue = (t0 * (d0 * d0) + t1 * (d1 * d1)) + t2 * (d2 * d2)
        return wm, ua, ue

    ca, cua, cue = mix(cls_ref[0], swap_l4=True)
    ca_ref[0] = ca
    cua_ref[0] = cua
    cue_ref[0] = cue
    ba, bua, bue = mix(box_ref[0])
    ba_ref[0] = ba
    bua_ref[0] = bua
    bue_ref[0] = bue


def _mix_all(cls_t, box_t):
    grid = (_B, _A // _SBLK)
    f32 = jnp.float32
    out_shape = [
        jax.ShapeDtypeStruct((_B, _A, _NC), f32),
        jax.ShapeDtypeStruct((_B, _A, _NC), f32),
        jax.ShapeDtypeStruct((_B, _A, _NC), f32),
        jax.ShapeDtypeStruct((_B, _A, 4), f32),
        jax.ShapeDtypeStruct((_B, _A, 4), f32),
        jax.ShapeDtypeStruct((_B, _A, 4), f32),
    ]
    in_specs = [
        pl.BlockSpec((1, 9, _SBLK, _NC), lambda b, s: (b, 0, s, 0)),
        pl.BlockSpec((1, 9, _SBLK, 4), lambda b, s: (b, 0, s, 0)),
    ]
    cspec = pl.BlockSpec((1, _SBLK, _NC), lambda b, s: (b, s, 0))
    bspec = pl.BlockSpec((1, _SBLK, 4), lambda b, s: (b, s, 0))
    out_specs = [cspec, cspec, cspec, bspec, bspec, bspec]
    return pl.pallas_call(
        _gmm_kernel,
        grid=grid,
        in_specs=in_specs,
        out_specs=out_specs,
        out_shape=out_shape,
    )(cls_t, box_t)


def _prep(levels, c):
    parts = [t.reshape(_B, 3, c, 3, -1) for t in levels]
    x = jnp.concatenate(parts, axis=-1)
    return x.transpose(0, 1, 3, 4, 2).reshape(_B, 9, _A, c)


def kernel(cls_0, cls_1, cls_2, cls_3, cls_4, box_0, box_1, box_2, box_3, box_4):
    cls_t = _prep([cls_0, cls_1, cls_2, cls_3, cls_4], _NC)
    box_t = _prep([box_0, box_1, box_2, box_3, box_4], 4)
    ca, cua, cue, ba, bua, bue = _mix_all(cls_t, box_t)

    _, topk_idx = jax.lax.top_k(ca.reshape(_B, _A * _NC), _K)
    indices = topk_idx // _NC
    classes = topk_idx % _NC

    def gbox(a):
        return jnp.take_along_axis(a, indices[:, :, None], axis=1)

    def gcls(a):
        g = jnp.take_along_axis(a, indices[:, :, None], axis=1)
        return jnp.take_along_axis(g, classes[:, :, None], axis=2)

    box_top = gbox(ba)
    box_ua_top = jnp.max(gbox(bua), axis=2, keepdims=True)
    box_ue_top = jnp.max(gbox(bue), axis=2, keepdims=True)
    cls_top = gcls(ca)
    cls_ua_top = jnp.max(gcls(cua), axis=2, keepdims=True)
    cls_ue_top = jnp.max(gcls(cue), axis=2, keepdims=True)
    return (cls_top, cls_ua_top, cls_ue_top, box_top, box_ua_top, box_ue_top,
            indices, classes)


# speedup vs baseline: 1.5605x; 1.5605x over previous
import jax
import jax.numpy as jnp
from jax.experimental import pallas as pl

_B = 2
_NC = 90
_A = 5456
_SBLK = 496
_K = 5000


def _gmm_kernel(cls_ref, box_ref, ca_ref, cua_ref, cue_ref, ba_ref, bua_ref, bue_ref):
    base = pl.program_id(1) * _SBLK

    def mix(x, swap_l4=False):
        m0, m1, m2 = x[0], x[1], x[2]
        v0, v1, v2 = x[3], x[4], x[5]
        w0, w1, w2 = x[6], x[7], x[8]
        mx = jnp.maximum(jnp.maximum(w0, w1), w2)
        e0 = jnp.exp(w0 - mx)
        e1 = jnp.exp(w1 - mx)
        e2 = jnp.exp(w2 - mx)
        s = (e0 + e1) + e2
        t0 = e0 / s
        t1 = e1 / s
        t2 = e2 / s
        p0, p1, p2 = t0 * m0, t1 * m1, t2 * m2
        wm = (p0 + p1) + p2
        if swap_l4:
            row = base + jax.lax.broadcasted_iota(jnp.int32, wm.shape, 0)
            wm = jnp.where(row >= _A - 16, (p0 + p2) + p1, wm)
        ua = (t0 * jax.nn.sigmoid(v0) + t1 * jax.nn.sigmoid(v1)) \
            + t2 * jax.nn.sigmoid(v2)
        d0 = m0 - wm
        d1 = m1 - wm
        d2 = m2 - wm
        ue = (t0 * (d0 * d0) + t1 * (d1 * d1)) + t2 * (d2 * d2)
        return wm, ua, ue

    ca, cua, cue = mix(cls_ref[0], swap_l4=True)
    ca_ref[0] = ca
    cua_ref[0] = cua
    cue_ref[0] = cue
    ba, bua, bue = mix(box_ref[0])
    ba_ref[0] = ba
    bua_ref[0] = bua
    bue_ref[0] = bue


def _mix_all(cls_t, box_t):
    grid = (_B, _A // _SBLK)
    f32 = jnp.float32
    out_shape = [
        jax.ShapeDtypeStruct((_B, _A, _NC), f32),
        jax.ShapeDtypeStruct((_B, _A, _NC), f32),
        jax.ShapeDtypeStruct((_B, _A, _NC), f32),
        jax.ShapeDtypeStruct((_B, _A, 4), f32),
        jax.ShapeDtypeStruct((_B, _A, 4), f32),
        jax.ShapeDtypeStruct((_B, _A, 4), f32),
    ]
    in_specs = [
        pl.BlockSpec((1, 9, _SBLK, _NC), lambda b, s: (b, 0, s, 0)),
        pl.BlockSpec((1, 9, _SBLK, 4), lambda b, s: (b, 0, s, 0)),
    ]
    cspec = pl.BlockSpec((1, _SBLK, _NC), lambda b, s: (b, s, 0))
    bspec = pl.BlockSpec((1, _SBLK, 4), lambda b, s: (b, s, 0))
    out_specs = [cspec, cspec, cspec, bspec, bspec, bspec]
    return pl.pallas_call(
        _gmm_kernel,
        grid=grid,
        in_specs=in_specs,
        out_specs=out_specs,
        out_shape=out_shape,
    )(cls_t, box_t)


_BUF = 8192
_NCHUNK = _A // 8


def _select_kernel(ca_ref, bv_ref, bi_ref):
    NEG = jnp.int32(-2147483648)
    BIGI = jnp.int32(2147483647)
    bv_ref[0] = jnp.full((_BUF, 128), -jnp.inf, jnp.float32)
    bi_ref[0] = jnp.full((_BUF, 128), BIGI, jnp.int32)

    def tokey(x):
        b = jax.lax.bitcast_convert_type(x, jnp.int32)
        return jnp.where(b >= 0, b, b ^ jnp.int32(2147483647))

    def bs_body(i, lohi):
        lo, hi = lohi
        x = lo ^ hi
        mid = (lo & hi) + (x >> 1) + (x & 1)
        cnt = jnp.sum((tokey(ca_ref[0]) >= mid).astype(jnp.int32))
        p = cnt >= _K
        return (jnp.where(p, mid, lo), jnp.where(p, hi, mid - jnp.int32(1)))

    tau, _ = jax.lax.fori_loop(0, 32, bs_body, (NEG, BIGI))

    def chunk_body(c, off):
        scores = ca_ref[0, pl.ds(c * 8, 8), :]
        flat = (c * 8 + jax.lax.broadcasted_iota(jnp.int32, (8, _NC), 0)) * _NC \
            + jax.lax.broadcasted_iota(jnp.int32, (8, _NC), 1)

        def get_m(keys_):
            return jnp.min(jnp.where(keys_ >= tau, flat, BIGI))

        keys0 = tokey(scores)

        def cond(st):
            off_, keys_, m_ = st
            return (m_ < BIGI) & (off_ < _BUF)

        def body(st):
            off_, keys_, m_ = st
            hit = flat == m_
            val = jnp.sum(jnp.where(hit, scores, 0.0))
            bv_ref[0, pl.ds(off_, 1), 0:1] = val.reshape(1, 1)
            bi_ref[0, pl.ds(off_, 1), 0:1] = m_.reshape(1, 1)
            keys2 = jnp.where(hit, NEG, keys_)
            return (off_ + jnp.int32(1), keys2, get_m(keys2))

        off_end, _, _ = jax.lax.while_loop(cond, body, (off, keys0, get_m(keys0)))
        return off_end

    jax.lax.fori_loop(0, _NCHUNK, chunk_body, jnp.int32(0))


def _select_topk(ca):
    bv, bi = pl.pallas_call(
        _select_kernel,
        grid=(_B,),
        in_specs=[pl.BlockSpec((1, _A, _NC), lambda b: (b, 0, 0))],
        out_specs=[pl.BlockSpec((1, _BUF, 128), lambda b: (b, 0, 0)),
                   pl.BlockSpec((1, _BUF, 128), lambda b: (b, 0, 0))],
        out_shape=[jax.ShapeDtypeStruct((_B, _BUF, 128), jnp.float32),
                   jax.ShapeDtypeStruct((_B, _BUF, 128), jnp.int32)],
    )(ca)
    vals = bv[:, :, 0]
    idxs = bi[:, :, 0]
    kk = jax.lax.bitcast_convert_type(vals, jnp.int32)
    keyn = ~jnp.where(kk >= 0, kk, kk ^ jnp.int32(2147483647))
    _, si, sv = jax.lax.sort((keyn, idxs, vals), num_keys=2)
    return sv[:, :_K], si[:, :_K]


def _prep(levels, c):
    parts = [t.reshape(_B, 3, c, 3, -1) for t in levels]
    x = jnp.concatenate(parts, axis=-1)
    return x.transpose(0, 1, 3, 4, 2).reshape(_B, 9, _A, c)


def kernel(cls_0, cls_1, cls_2, cls_3, cls_4, box_0, box_1, box_2, box_3, box_4):
    cls_t = _prep([cls_0, cls_1, cls_2, cls_3, cls_4], _NC)
    box_t = _prep([box_0, box_1, box_2, box_3, box_4], 4)
    ca, cua, cue, ba, bua, bue = _mix_all(cls_t, box_t)

    topv, topk_idx = _select_topk(ca)
    indices = topk_idx // _NC
    classes = topk_idx % _NC

    def gbox(a):
        return jnp.take_along_axis(a, indices[:, :, None], axis=1)

    def gcls(a):
        g = jnp.take_along_axis(a, indices[:, :, None], axis=1)
        return jnp.take_along_axis(g, classes[:, :, None], axis=2)

    box_top = gbox(ba)
    box_ua_top = jnp.max(gbox(bua), axis=2, keepdims=True)
    box_ue_top = jnp.max(gbox(bue), axis=2, keepdims=True)
    cls_top = topv[:, :, None]
    cls_ua_top = jnp.max(gcls(cua), axis=2, keepdims=True)
    cls_ue_top = jnp.max(gcls(cue), axis=2, keepdims=True)
    return (cls_top, cls_ua_top, cls_ue_top, box_top, box_ua_top, box_ue_top,
            indices, classes)


# speedup vs baseline: 3.6194x; 2.3194x over previous
import jax
import jax.numpy as jnp
from jax.experimental import pallas as pl

_B = 2
_NC = 90
_A = 5456
_SBLK = 496
_K = 5000


def _gmm_kernel(cls_ref, box_ref, ca_ref, cua_ref, cue_ref, ba_ref, bua_ref, bue_ref):
    base = pl.program_id(1) * _SBLK

    def mix(x, swap_l4=False):
        m0, m1, m2 = x[0], x[1], x[2]
        v0, v1, v2 = x[3], x[4], x[5]
        w0, w1, w2 = x[6], x[7], x[8]
        mx = jnp.maximum(jnp.maximum(w0, w1), w2)
        e0 = jnp.exp(w0 - mx)
        e1 = jnp.exp(w1 - mx)
        e2 = jnp.exp(w2 - mx)
        s = (e0 + e1) + e2
        t0 = e0 / s
        t1 = e1 / s
        t2 = e2 / s
        p0, p1, p2 = t0 * m0, t1 * m1, t2 * m2
        wm = (p0 + p1) + p2
        if swap_l4:
            row = base + jax.lax.broadcasted_iota(jnp.int32, wm.shape, 0)
            wm = jnp.where(row >= _A - 16, (p0 + p2) + p1, wm)
        ua = (t0 * jax.nn.sigmoid(v0) + t1 * jax.nn.sigmoid(v1)) \
            + t2 * jax.nn.sigmoid(v2)
        d0 = m0 - wm
        d1 = m1 - wm
        d2 = m2 - wm
        ue = (t0 * (d0 * d0) + t1 * (d1 * d1)) + t2 * (d2 * d2)
        return wm, ua, ue

    ca, cua, cue = mix(cls_ref[0], swap_l4=True)
    ca_ref[0] = ca
    cua_ref[0] = cua
    cue_ref[0] = cue
    ba, bua, bue = mix(box_ref[0])
    ba_ref[0] = ba
    bua_ref[0] = bua
    bue_ref[0] = bue


def _mix_all(cls_t, box_t):
    grid = (_B, _A // _SBLK)
    f32 = jnp.float32
    out_shape = [
        jax.ShapeDtypeStruct((_B, _A, _NC), f32),
        jax.ShapeDtypeStruct((_B, _A, _NC), f32),
        jax.ShapeDtypeStruct((_B, _A, _NC), f32),
        jax.ShapeDtypeStruct((_B, _A, 4), f32),
        jax.ShapeDtypeStruct((_B, _A, 4), f32),
        jax.ShapeDtypeStruct((_B, _A, 4), f32),
    ]
    in_specs = [
        pl.BlockSpec((1, 9, _SBLK, _NC), lambda b, s: (b, 0, s, 0)),
        pl.BlockSpec((1, 9, _SBLK, 4), lambda b, s: (b, 0, s, 0)),
    ]
    cspec = pl.BlockSpec((1, _SBLK, _NC), lambda b, s: (b, s, 0))
    bspec = pl.BlockSpec((1, _SBLK, 4), lambda b, s: (b, s, 0))
    out_specs = [cspec, cspec, cspec, bspec, bspec, bspec]
    return pl.pallas_call(
        _gmm_kernel,
        grid=grid,
        in_specs=in_specs,
        out_specs=out_specs,
        out_shape=out_shape,
    )(cls_t, box_t)


_BUF = 8192
_NCHUNK = _A // 8


_NPEEL = 4


def _select_kernel(ca_ref, pv_ref, pi_ref, fv_ref, fi_ref, kref):
    NEG = jnp.int32(-2147483648)
    BIGI = jnp.int32(2147483647)
    fv_ref[0] = jnp.full((_BUF, 128), -jnp.inf, jnp.float32)
    fi_ref[0] = jnp.full((_BUF, 128), BIGI, jnp.int32)

    def tokey(x):
        b = jax.lax.bitcast_convert_type(x, jnp.int32)
        return jnp.where(b >= 0, b, b ^ jnp.int32(2147483647))

    def unkey(k):
        return jax.lax.bitcast_convert_type(
            jnp.where(k >= 0, k, k ^ jnp.int32(2147483647)), jnp.float32)

    kref[...] = tokey(ca_ref[0])

    def bs_body(i, lohi):
        lo, hi = lohi
        x = lo ^ hi
        mid = (lo & hi) + (x >> 1) + (x & 1)
        cnt = jnp.sum((kref[...] >= mid).astype(jnp.int32))
        p = cnt >= _K
        return (jnp.where(p, mid, lo), jnp.where(p, hi, mid - jnp.int32(1)))

    tau, _ = jax.lax.fori_loop(0, 32, bs_body, (NEG, BIGI))

    ciota = jax.lax.broadcasted_iota(jnp.int32, (_A, _NC), 1)
    riota = jax.lax.broadcasted_iota(jnp.int32, (_A, 1), 0)
    for p in range(_NPEEL):
        keys = kref[...]
        masked = jnp.where(keys >= tau, keys, NEG)
        rowbest = jnp.max(masked, axis=1, keepdims=True)
        hasrow = rowbest > NEG
        hit = masked == rowbest
        firstc = jnp.min(jnp.where(hit & hasrow, ciota, BIGI), axis=1,
                         keepdims=True)
        taken = hasrow & (ciota == firstc)
        pv_ref[0, :, p:p + 1] = jnp.where(hasrow, unkey(rowbest), -jnp.inf)
        pi_ref[0, :, p:p + 1] = jnp.where(hasrow, riota * _NC + firstc, BIGI)
        kref[...] = jnp.where(taken, NEG, keys)

    def chunk_body(c, off):
        flat = (c * 8 + jax.lax.broadcasted_iota(jnp.int32, (8, _NC), 0)) * _NC \
            + jax.lax.broadcasted_iota(jnp.int32, (8, _NC), 1)
        keys0 = kref[pl.ds(c * 8, 8), :]

        def get_m(keys_):
            return jnp.min(jnp.where(keys_ >= tau, flat, BIGI))

        def cond(st):
            off_, keys_, m_ = st
            return (m_ < BIGI) & (off_ < _BUF)

        def body(st):
            off_, keys_, m_ = st
            hit_ = flat == m_
            kbit = jnp.sum(jnp.where(hit_, keys_, 0))
            fv_ref[0, pl.ds(off_, 1), 0:1] = unkey(kbit).reshape(1, 1)
            fi_ref[0, pl.ds(off_, 1), 0:1] = m_.reshape(1, 1)
            keys2 = jnp.where(hit_, NEG, keys_)
            return (off_ + jnp.int32(1), keys2, get_m(keys2))

        off_end, _, _ = jax.lax.while_loop(cond, body, (off, keys0, get_m(keys0)))
        return off_end

    jax.lax.fori_loop(0, _NCHUNK, chunk_body, jnp.int32(0))


def _select_topk(ca):
    from jax.experimental.pallas import tpu as pltpu
    pv, pi, fv, fi = pl.pallas_call(
        _select_kernel,
        grid=(_B,),
        in_specs=[pl.BlockSpec((1, _A, _NC), lambda b: (b, 0, 0))],
        out_specs=[pl.BlockSpec((1, _A, 128), lambda b: (b, 0, 0)),
                   pl.BlockSpec((1, _A, 128), lambda b: (b, 0, 0)),
                   pl.BlockSpec((1, _BUF, 128), lambda b: (b, 0, 0)),
                   pl.BlockSpec((1, _BUF, 128), lambda b: (b, 0, 0))],
        out_shape=[jax.ShapeDtypeStruct((_B, _A, 128), jnp.float32),
                   jax.ShapeDtypeStruct((_B, _A, 128), jnp.int32),
                   jax.ShapeDtypeStruct((_B, _BUF, 128), jnp.float32),
                   jax.ShapeDtypeStruct((_B, _BUF, 128), jnp.int32)],
        scratch_shapes=[pltpu.VMEM((_A, _NC), jnp.int32)],
    )(ca)
    vals = jnp.concatenate(
        [pv[:, :, :_NPEEL].reshape(_B, -1), fv[:, :, 0]], axis=1)
    idxs = jnp.concatenate(
        [pi[:, :, :_NPEEL].reshape(_B, -1), fi[:, :, 0]], axis=1)
    kk = jax.lax.bitcast_convert_type(vals, jnp.int32)
    keyn = ~jnp.where(kk >= 0, kk, kk ^ jnp.int32(2147483647))
    _, si, sv = jax.lax.sort((keyn, idxs, vals), num_keys=2)
    return sv[:, :_K], si[:, :_K]


def _prep(levels, c):
    parts = [t.reshape(_B, 3, c, 3, -1) for t in levels]
    x = jnp.concatenate(parts, axis=-1)
    return x.transpose(0, 1, 3, 4, 2).reshape(_B, 9, _A, c)


def kernel(cls_0, cls_1, cls_2, cls_3, cls_4, box_0, box_1, box_2, box_3, box_4):
    cls_t = _prep([cls_0, cls_1, cls_2, cls_3, cls_4], _NC)
    box_t = _prep([box_0, box_1, box_2, box_3, box_4], 4)
    ca, cua, cue, ba, bua, bue = _mix_all(cls_t, box_t)

    topv, topk_idx = _select_topk(ca)
    indices = topk_idx // _NC
    classes = topk_idx % _NC

    def gbox(a):
        return jnp.take_along_axis(a, indices[:, :, None], axis=1)

    def gcls(a):
        g = jnp.take_along_axis(a, indices[:, :, None], axis=1)
        return jnp.take_along_axis(g, classes[:, :, None], axis=2)

    box_top = gbox(ba)
    box_ua_top = jnp.max(gbox(bua), axis=2, keepdims=True)
    box_ue_top = jnp.max(gbox(bue), axis=2, keepdims=True)
    cls_top = topv[:, :, None]
    cls_ua_top = jnp.max(gcls(cua), axis=2, keepdims=True)
    cls_ue_top = jnp.max(gcls(cue), axis=2, keepdims=True)
    return (cls_top, cls_ua_top, cls_ue_top, box_top, box_ua_top, box_ue_top,
            indices, classes)
